# Initial kernel scaffold; baseline (speedup 1.0000x reference)
import jax
import jax.numpy as jnp
from jax.experimental import pallas as pl


def kernel(center_words, context_words, negative_words, input_embeddings, output_embeddings):
    raise NotImplementedError("write your pallas kernel here")


# speedup vs baseline: 3.8978x; 3.8978x over previous
import functools

import jax
import jax.numpy as jnp
from jax import lax
from jax.experimental import pallas as pl
from jax.experimental.pallas import tpu as pltpu
from jax.experimental.pallas import tpu_sc as plsc

D = 64
K = 20
NC = 2
NS = 16
NW = NC * NS
LANES = 16

CHUNK_B = 32
ROWS_PER_CHUNK = CHUNK_B * K
IDX_DMA = 128


def _sc_body(batch, cw_hbm, xw_hbm, nw_hbm, ie_hbm, oe_hbm,
             pos_hbm, neg_hbm,
             idx_c, idx_x, idx_n, cen, ctx, negb, pos_v, neg_v, sem):
    nb = batch // NW
    wid = lax.axis_index("s") * NC + lax.axis_index("c")
    base = wid * nb

    n_idx_rows = nb // IDX_DMA
    for j in range(n_idx_rows):
        pltpu.sync_copy(cw_hbm.at[pl.ds(base + j * IDX_DMA, IDX_DMA)],
                        idx_c.at[j])
        pltpu.sync_copy(xw_hbm.at[pl.ds(base + j * IDX_DMA, IDX_DMA)],
                        idx_x.at[j])
    cps = []
    for j in range(n_idx_rows):
        cps.append(pltpu.async_copy(
            ie_hbm.at[idx_c.at[j]], cen.at[pl.ds(j * IDX_DMA, IDX_DMA)], sem))
        cps.append(pltpu.async_copy(
            oe_hbm.at[idx_x.at[j]], ctx.at[pl.ds(j * IDX_DMA, IDX_DMA)], sem))
    for cp in cps:
        cp.wait()

    iota = lax.iota(jnp.int32, LANES)
    n_chunks = nb // CHUNK_B
    groups_per_chunk = CHUNK_B // LANES

    def group_body(c, g):
        b0 = c * CHUNK_B + g * LANES
        b_vec = b0 + iota
        r0 = g * LANES * K
        rvecs = [r0 + K * iota + k for k in range(K)]
        zero = jnp.zeros((LANES,), jnp.float32)

        def d_body(d, carry):
            dv = jnp.full((LANES,), d, jnp.int32)
            cd = plsc.load_gather(cen, [b_vec, dv])
            xd = plsc.load_gather(ctx, [b_vec, dv])
            accp = carry[0] + cd * xd
            accs = tuple(
                carry[1 + k] + cd * plsc.load_gather(negb, [rvecs[k], dv])
                for k in range(K))
            return (accp,) + accs

        res = lax.fori_loop(0, D, d_body, (zero,) * (K + 1))
        pos_v[pl.ds(b0, LANES)] = res[0]
        for k in range(K):
            neg_v[k, pl.ds(b0, LANES)] = res[1 + k]

    def chunk_body(c, _):
        off = base * K + c * ROWS_PER_CHUNK
        for j in range(ROWS_PER_CHUNK // IDX_DMA):
            pltpu.sync_copy(nw_hbm.at[pl.ds(off + j * IDX_DMA, IDX_DMA)],
                            idx_n.at[j])
        ncps = [pltpu.async_copy(oe_hbm.at[idx_n.at[j]],
                                 negb.at[pl.ds(j * IDX_DMA, IDX_DMA)], sem)
                for j in range(ROWS_PER_CHUNK // IDX_DMA)]
        for cp in ncps:
            cp.wait()
        return lax.fori_loop(
            0, groups_per_chunk,
            lambda g, carry: (group_body(c, g), carry)[1], None)

    lax.fori_loop(0, n_chunks, chunk_body, None)

    pltpu.sync_copy(pos_v, pos_hbm.at[pl.ds(base, nb)])
    pltpu.sync_copy(neg_v, neg_hbm.at[wid])


def _sc_scores(cw, xw, nw_flat, ie, oe):
    batch = cw.shape[0]
    nb = batch // NW
    mesh = plsc.VectorSubcoreMesh(core_axis_name="c", subcore_axis_name="s",
                                  num_cores=NC, num_subcores=NS)
    f = pl.kernel(
        functools.partial(_sc_body, batch),
        out_type=(jax.ShapeDtypeStruct((batch,), jnp.float32),
                  jax.ShapeDtypeStruct((NW, K, nb), jnp.float32)),
        mesh=mesh,
        compiler_params=pltpu.CompilerParams(
            needs_layout_passes=False, use_tc_tiling_on_sc=False),
        scratch_types=[
            pltpu.VMEM((nb // IDX_DMA, IDX_DMA), jnp.int32),
            pltpu.VMEM((nb // IDX_DMA, IDX_DMA), jnp.int32),
            pltpu.VMEM((ROWS_PER_CHUNK // IDX_DMA, IDX_DMA), jnp.int32),
            pltpu.VMEM((nb, D), jnp.float32),
            pltpu.VMEM((nb, D), jnp.float32),
            pltpu.VMEM((ROWS_PER_CHUNK, D), jnp.float32),
            pltpu.VMEM((nb,), jnp.float32),
            pltpu.VMEM((K, nb), jnp.float32),
            pltpu.SemaphoreType.DMA,
        ],
    )
    return f(cw, xw, nw_flat, ie, oe)


def _finish_body(batch, pos_ref, neg_ref, out_ref):
    pos = pos_ref[...]
    neg = neg_ref[...]

    def logsig(x):
        return jnp.minimum(x, 0.0) - jnp.log1p(jnp.exp(-jnp.abs(x)))

    total = jnp.sum(logsig(pos)) + jnp.sum(logsig(-neg))
    out_ref[0, 0] = -total / batch


def _finish(pos2d, neg2d, batch):
    return pl.pallas_call(
        functools.partial(_finish_body, batch),
        out_shape=jax.ShapeDtypeStruct((1, 1), jnp.float32),
        out_specs=pl.BlockSpec(memory_space=pltpu.SMEM),
    )(pos2d, neg2d)


def kernel(center_words, context_words, negative_words,
           input_embeddings, output_embeddings):
    batch = center_words.shape[0]
    cw = center_words.astype(jnp.int32)
    xw = context_words.astype(jnp.int32)
    nw_flat = negative_words.astype(jnp.int32).reshape(-1)
    pos, neg = _sc_scores(cw, xw, nw_flat,
                          input_embeddings, output_embeddings)
    out = _finish(pos.reshape(NW, batch // NW),
                  neg.reshape(NW * K, batch // NW), batch)
    return out.reshape(())


# speedup vs baseline: 4.7841x; 1.2274x over previous
import functools

import jax
import jax.numpy as jnp
from jax import lax
from jax.experimental import pallas as pl
from jax.experimental.pallas import tpu as pltpu
from jax.experimental.pallas import tpu_sc as plsc

D = 64
K = 20
NC = 2
NS = 16
NW = NC * NS
LANES = 16

CHUNK_B = 32
ROWS_PER_CHUNK = CHUNK_B * K
IDX_DMA = 128


def _sc_body(batch, cw_hbm, xw_hbm, nw_hbm, ie_hbm, oe_hbm,
             pos_hbm, neg_hbm,
             idx_c, idx_x, idx_n, cen, ctx, negb, pos_v, neg_v, sem):
    nb = batch // NW
    wid = lax.axis_index("s") * NC + lax.axis_index("c")
    base = wid * nb

    n_idx_rows = nb // IDX_DMA
    for j in range(n_idx_rows):
        pltpu.sync_copy(cw_hbm.at[pl.ds(base + j * IDX_DMA, IDX_DMA)],
                        idx_c.at[j])
        pltpu.sync_copy(xw_hbm.at[pl.ds(base + j * IDX_DMA, IDX_DMA)],
                        idx_x.at[j])
    cps = []
    for j in range(n_idx_rows):
        cps.append(pltpu.async_copy(
            ie_hbm.at[idx_c.at[j]], cen.at[pl.ds(j * IDX_DMA, IDX_DMA)], sem))
        cps.append(pltpu.async_copy(
            oe_hbm.at[idx_x.at[j]], ctx.at[pl.ds(j * IDX_DMA, IDX_DMA)], sem))
    for cp in cps:
        cp.wait()

    iota = lax.iota(jnp.int32, LANES)
    n_chunks = nb // CHUNK_B
    groups_per_chunk = CHUNK_B // LANES

    def group_body(c, g):
        b0 = c * CHUNK_B + g * LANES
        b_vec = b0 + iota
        r0 = g * LANES * K
        rvecs = [r0 + K * iota + k for k in range(K)]
        zero = jnp.zeros((LANES,), jnp.float32)
        diag = 8 * iota

        def d_body(d, carry):
            dv = (diag + d) & (D - 1)
            cd = plsc.load_gather(cen, [b_vec, dv])
            xd = plsc.load_gather(ctx, [b_vec, dv])
            accp = carry[0] + cd * xd
            accs = tuple(
                carry[1 + k] + cd * plsc.load_gather(negb, [rvecs[k], dv])
                for k in range(K))
            return (accp,) + accs

        res = lax.fori_loop(0, D, d_body, (zero,) * (K + 1))
        pos_v[pl.ds(b0, LANES)] = res[0]
        for k in range(K):
            neg_v[k, pl.ds(b0, LANES)] = res[1 + k]

    def chunk_body(c, _):
        off = base * K + c * ROWS_PER_CHUNK
        for j in range(ROWS_PER_CHUNK // IDX_DMA):
            pltpu.sync_copy(nw_hbm.at[pl.ds(off + j * IDX_DMA, IDX_DMA)],
                            idx_n.at[j])
        ncps = [pltpu.async_copy(oe_hbm.at[idx_n.at[j]],
                                 negb.at[pl.ds(j * IDX_DMA, IDX_DMA)], sem)
                for j in range(ROWS_PER_CHUNK // IDX_DMA)]
        for cp in ncps:
            cp.wait()
        return lax.fori_loop(
            0, groups_per_chunk,
            lambda g, carry: (group_body(c, g), carry)[1], None)

    lax.fori_loop(0, n_chunks, chunk_body, None)

    pltpu.sync_copy(pos_v, pos_hbm.at[pl.ds(base, nb)])
    pltpu.sync_copy(neg_v, neg_hbm.at[wid])


def _sc_scores(cw, xw, nw_flat, ie, oe):
    batch = cw.shape[0]
    nb = batch // NW
    mesh = plsc.VectorSubcoreMesh(core_axis_name="c", subcore_axis_name="s",
                                  num_cores=NC, num_subcores=NS)
    f = pl.kernel(
        functools.partial(_sc_body, batch),
        out_type=(jax.ShapeDtypeStruct((batch,), jnp.float32),
                  jax.ShapeDtypeStruct((NW, K, nb), jnp.float32)),
        mesh=mesh,
        compiler_params=pltpu.CompilerParams(
            needs_layout_passes=False, use_tc_tiling_on_sc=False),
        scratch_types=[
            pltpu.VMEM((nb // IDX_DMA, IDX_DMA), jnp.int32),
            pltpu.VMEM((nb // IDX_DMA, IDX_DMA), jnp.int32),
            pltpu.VMEM((ROWS_PER_CHUNK // IDX_DMA, IDX_DMA), jnp.int32),
            pltpu.VMEM((nb, D), jnp.float32),
            pltpu.VMEM((nb, D), jnp.float32),
            pltpu.VMEM((ROWS_PER_CHUNK, D), jnp.float32),
            pltpu.VMEM((nb,), jnp.float32),
            pltpu.VMEM((K, nb), jnp.float32),
            pltpu.SemaphoreType.DMA,
        ],
    )
    return f(cw, xw, nw_flat, ie, oe)


def _finish_body(batch, pos_ref, neg_ref, out_ref):
    pos = pos_ref[...]
    neg = neg_ref[...]

    def logsig(x):
        return jnp.minimum(x, 0.0) - jnp.log1p(jnp.exp(-jnp.abs(x)))

    total = jnp.sum(logsig(pos)) + jnp.sum(logsig(-neg))
    out_ref[0, 0] = -total / batch


def _finish(pos2d, neg2d, batch):
    return pl.pallas_call(
        functools.partial(_finish_body, batch),
        out_shape=jax.ShapeDtypeStruct((1, 1), jnp.float32),
        out_specs=pl.BlockSpec(memory_space=pltpu.SMEM),
    )(pos2d, neg2d)


def kernel(center_words, context_words, negative_words,
           input_embeddings, output_embeddings):
    batch = center_words.shape[0]
    cw = center_words.astype(jnp.int32)
    xw = context_words.astype(jnp.int32)
    nw_flat = negative_words.astype(jnp.int32).reshape(-1)
    pos, neg = _sc_scores(cw, xw, nw_flat,
                          input_embeddings, output_embeddings)
    out = _finish(pos.reshape(NW, batch // NW),
                  neg.reshape(NW * K, batch // NW), batch)
    return out.reshape(())
